# baseline (speedup 1.0000x reference)
import functools

import jax
import jax.numpy as jnp
from jax import lax
from jax.experimental import pallas as pl
from jax.experimental.pallas import tpu as pltpu
from jax.experimental.pallas import tpu_sc as plsc

NUM_EXPERTS = 8
TOP_K = 2
HIDDEN = 1024
INTER = 512
N_GROUP = 4
GSZ = NUM_EXPERTS // N_GROUP
TOPK_GROUP = 2
ROUTED_SCALE = 2.5
TOKENS = 2048

TBLK = 256
B = 128
NT = 40
NTP = 48
NR = NT * B

NC = 2
NS = 16
NW = NC * NS
TPW = TOKENS // NW
TPS = TOKENS // NS
RPW = NR // NW
GCH = 32

_INTERP = False


def _routing_body(x_ref, gwt_ref, gb_ref, scores_ref, mask_ref):
    x = x_ref[...]
    logits = jnp.dot(x, gwt_ref[...], preferred_element_type=jnp.float32)
    s = jax.nn.sigmoid(logits)
    swb = s + gb_ref[...]

    col = lambda a, i: a[:, i : i + 1]
    g = [sum(col(swb, gi * GSZ + j) for j in range(GSZ)) for gi in range(N_GROUP)]
    gsel = []
    for gi in range(N_GROUP):
        rank = jnp.zeros_like(g[gi])
        for gj in range(N_GROUP):
            if gj == gi:
                continue
            beats = g[gj] > g[gi]
            if gj < gi:
                beats = beats | (g[gj] == g[gi])
            rank = rank + beats.astype(jnp.float32)
        gsel.append(rank < TOPK_GROUP)
    swbm = [jnp.where(gsel[e // GSZ], col(swb, e), 0.0) for e in range(NUM_EXPERTS)]
    sel = []
    for e in range(NUM_EXPERTS):
        rank = jnp.zeros_like(swbm[e])
        for e2 in range(NUM_EXPERTS):
            if e2 == e:
                continue
            beats = swbm[e2] > swbm[e]
            if e2 < e:
                beats = beats | (swbm[e2] == swbm[e])
            rank = rank + beats.astype(jnp.float32)
        sel.append(rank < TOP_K)
    sc = [jnp.where(sel[e], col(s, e), 0.0) for e in range(NUM_EXPERTS)]
    denom = sum(sc) + 1e-20
    w = [sc[e] / denom * ROUTED_SCALE for e in range(NUM_EXPERTS)]
    scores_ref[...] = jnp.concatenate(w, axis=1)
    mask_ref[...] = jnp.concatenate(
        [sel[e].astype(jnp.int32) for e in range(NUM_EXPERTS)], axis=1
    )


def _routing(x, gate_weight, gate_bias):
    nblk = TOKENS // TBLK
    return pl.pallas_call(
        _routing_body,
        grid=(nblk,),
        in_specs=[
            pl.BlockSpec((TBLK, HIDDEN), lambda t: (t, 0)),
            pl.BlockSpec((HIDDEN, NUM_EXPERTS), lambda t: (0, 0)),
            pl.BlockSpec((1, NUM_EXPERTS), lambda t: (0, 0)),
        ],
        out_specs=[
            pl.BlockSpec((TBLK, NUM_EXPERTS), lambda t: (t, 0)),
            pl.BlockSpec((TBLK, NUM_EXPERTS), lambda t: (t, 0)),
        ],
        out_shape=[
            jax.ShapeDtypeStruct((TOKENS, NUM_EXPERTS), jnp.float32),
            jax.ShapeDtypeStruct((TOKENS, NUM_EXPERTS), jnp.int32),
        ],
        interpret=_INTERP,
    )(x, gate_weight.T, gate_bias.reshape(1, NUM_EXPERTS))


def _shared_body(x_ref, wg_ref, wu_ref, wd_ref, out_ref):
    x = x_ref[...]
    hg = jnp.dot(x, wg_ref[...], preferred_element_type=jnp.float32)
    hu = jnp.dot(x, wu_ref[...], preferred_element_type=jnp.float32)
    h = jax.nn.silu(hg) * hu
    out_ref[...] = jnp.dot(h, wd_ref[...], preferred_element_type=jnp.float32)


def _shared(x, wg, wu, wd):
    nblk = TOKENS // TBLK
    return pl.pallas_call(
        _shared_body,
        grid=(nblk,),
        in_specs=[
            pl.BlockSpec((TBLK, HIDDEN), lambda t: (t, 0)),
            pl.BlockSpec((HIDDEN, INTER), lambda t: (0, 0)),
            pl.BlockSpec((HIDDEN, INTER), lambda t: (0, 0)),
            pl.BlockSpec((INTER, HIDDEN), lambda t: (0, 0)),
        ],
        out_specs=pl.BlockSpec((TBLK, HIDDEN), lambda t: (t, 0)),
        out_shape=jax.ShapeDtypeStruct((TOKENS, HIDDEN), jnp.float32),
        interpret=_INTERP,
    )(x, wg, wu, wd)


def _iota16():
    return lax.iota(jnp.int32, 16)


def _splat_i(s):
    return jnp.full((16,), s, jnp.int32)


def _scan_gather_body(x_hbm, mask_hbm, scores_hbm,
                      xs_hbm, wsort_hbm, pos0_hbm, pos1_hbm, te_hbm,
                      mvm, svm, cntv, partials_v, partials_sh,
                      p0loc, p1loc, w0loc, w1loc,
                      p0sh, p1sh, w0sh, w1sh,
                      p0all, p1all, w0all, w1all,
                      stloc, wsloc, teloc, rowbuf, sem):
    cid = lax.axis_index("c")
    sid = lax.axis_index("s")
    wid = cid * NS + sid
    t0 = sid * TPS

    pltpu.sync_copy(mask_hbm.at[pl.ds(t0 * NUM_EXPERTS, TPS * NUM_EXPERTS)], mvm)
    pltpu.sync_copy(scores_hbm.at[pl.ds(t0 * NUM_EXPERTS, TPS * NUM_EXPERTS)], svm)

    it16 = _iota16()
    cnt_vec = jnp.zeros((16,), jnp.int32)
    for e in range(NUM_EXPERTS):
        acc = jnp.zeros((16,), jnp.int32)
        for ch in range(TPS // 16):
            fidx = (it16 + (ch * 16)) * NUM_EXPERTS + e
            acc = acc + plsc.load_gather(mvm, [fidx])
        cnt_e = jnp.sum(acc)
        cnt_vec = cnt_vec + jnp.where(it16 == e, cnt_e, 0)
    cntv[...] = cnt_vec
    pltpu.sync_copy(cntv, partials_sh.at[sid])
    plsc.subcore_barrier()

    pltpu.sync_copy(partials_sh, partials_v)
    base_vec = jnp.zeros((16,), jnp.int32)
    totc_vec = jnp.zeros((16,), jnp.int32)
    for s2 in range(NS):
        row = partials_v[s2]
        pred = jnp.full((16,), s2 < sid)
        base_vec = base_vec + jnp.where(pred, row, 0)
        totc_vec = totc_vec + row
    tiles_vec = (totc_vec + (B - 1)) // B
    cumt_vec = jnp.cumsum(tiles_vec)
    seg_vec = (cumt_vec - tiles_vec) * B

    carry = [seg_vec[e] + base_vec[e] for e in range(NUM_EXPERTS)]
    for ch in range(TPS // 16):
        tokidx = it16 + (ch * 16)
        acc_k = jnp.zeros((16,), jnp.int32)
        p0v = jnp.zeros((16,), jnp.int32)
        p1v = jnp.zeros((16,), jnp.int32)
        w0v = jnp.zeros((16,), jnp.float32)
        w1v = jnp.zeros((16,), jnp.float32)
        for e in range(NUM_EXPERTS):
            fidx = tokidx * NUM_EXPERTS + e
            mv = plsc.load_gather(mvm, [fidx])
            sv = plsc.load_gather(svm, [fidx])
            excl = jnp.cumsum(mv) - mv
            posv = excl + carry[e]
            selb = mv > 0
            first = selb & (acc_k == 0)
            second = selb & (acc_k == 1)
            p0v = jnp.where(first, posv, p0v)
            w0v = jnp.where(first, sv, w0v)
            p1v = jnp.where(second, posv, p1v)
            w1v = jnp.where(second, sv, w1v)
            acc_k = acc_k + mv
            carry[e] = carry[e] + jnp.sum(mv)
        p0loc[pl.ds(ch * 16, 16)] = p0v
        p1loc[pl.ds(ch * 16, 16)] = p1v
        w0loc[pl.ds(ch * 16, 16)] = w0v
        w1loc[pl.ds(ch * 16, 16)] = w1v

    pltpu.sync_copy(p0loc, p0sh.at[pl.ds(t0, TPS)])
    pltpu.sync_copy(p1loc, p1sh.at[pl.ds(t0, TPS)])
    pltpu.sync_copy(w0loc, w0sh.at[pl.ds(t0, TPS)])
    pltpu.sync_copy(w1loc, w1sh.at[pl.ds(t0, TPS)])

    @pl.when(cid == 0)
    def _():
        pltpu.sync_copy(p0loc, pos0_hbm.at[pl.ds(t0, TPS)])

    @pl.when(cid == 1)
    def _():
        pltpu.sync_copy(p1loc, pos1_hbm.at[pl.ds(t0, TPS)])

    @pl.when(wid == 0)
    def _():
        for ch in range(NTP // 16):
            jv = it16 + (ch * 16)
            te = jnp.zeros((16,), jnp.int32)
            for e in range(NUM_EXPERTS):
                te = te + (jv >= cumt_vec[e]).astype(jnp.int32)
            teloc[pl.ds(ch * 16, 16)] = jnp.minimum(te, NUM_EXPERTS - 1)
        pltpu.sync_copy(teloc, te_hbm)

    plsc.subcore_barrier()

    pltpu.sync_copy(p0sh, p0all)
    pltpu.sync_copy(p1sh, p1all)
    pltpu.sync_copy(w0sh, w0all)
    pltpu.sync_copy(w1sh, w1all)

    r0 = wid * RPW
    zi = jnp.zeros((16,), jnp.int32)
    zf = jnp.zeros((16,), jnp.float32)
    for ch in range(RPW // 16):
        stloc[pl.ds(ch * 16, 16)] = zi
        wsloc[pl.ds(ch * 16, 16)] = zf
    for ch in range(TOKENS // 16):
        sl = pl.ds(ch * 16, 16)
        tokv = it16 + (ch * 16)
        for pall, wall in ((p0all, w0all), (p1all, w1all)):
            pv = pall[sl] - r0
            wv = wall[sl]
            inb = (pv >= 0) & (pv < RPW)
            plsc.store_scatter(stloc, [pv], tokv, mask=inb)
            plsc.store_scatter(wsloc, [pv], wv, mask=inb)

    pltpu.sync_copy(wsloc, wsort_hbm.at[pl.ds(r0, RPW)])

    for ch in range(RPW // GCH):
        idx = stloc.at[pl.ds(ch * GCH, GCH)]
        pltpu.async_copy(x_hbm.at[idx], rowbuf, sem).wait()
        pltpu.sync_copy(rowbuf, xs_hbm.at[pl.ds(r0 + ch * GCH, GCH)])


def _scan_gather(x, mask2, scores2):
    mesh = plsc.VectorSubcoreMesh(core_axis_name="c", subcore_axis_name="s")
    kern = functools.partial(
        pl.kernel,
        mesh=mesh,
        compiler_params=pltpu.CompilerParams(needs_layout_passes=False),
        out_type=[
            jax.ShapeDtypeStruct((NR, HIDDEN), jnp.float32),
            jax.ShapeDtypeStruct((NR,), jnp.float32),
            jax.ShapeDtypeStruct((TOKENS,), jnp.int32),
            jax.ShapeDtypeStruct((TOKENS,), jnp.int32),
            jax.ShapeDtypeStruct((NTP,), jnp.int32),
        ],
        scratch_types=[
            pltpu.VMEM((TPS * NUM_EXPERTS,), jnp.int32),
            pltpu.VMEM((TPS * NUM_EXPERTS,), jnp.float32),
            pltpu.VMEM((16,), jnp.int32),
            pltpu.VMEM((NS, 16), jnp.int32),
            pltpu.VMEM_SHARED((NS, 16), jnp.int32),
            pltpu.VMEM((TPS,), jnp.int32),
            pltpu.VMEM((TPS,), jnp.int32),
            pltpu.VMEM((TPS,), jnp.float32),
            pltpu.VMEM((TPS,), jnp.float32),
            pltpu.VMEM_SHARED((TOKENS,), jnp.int32),
            pltpu.VMEM_SHARED((TOKENS,), jnp.int32),
            pltpu.VMEM_SHARED((TOKENS,), jnp.float32),
            pltpu.VMEM_SHARED((TOKENS,), jnp.float32),
            pltpu.VMEM((TOKENS,), jnp.int32),
            pltpu.VMEM((TOKENS,), jnp.int32),
            pltpu.VMEM((TOKENS,), jnp.float32),
            pltpu.VMEM((TOKENS,), jnp.float32),
            pltpu.VMEM((RPW,), jnp.int32),
            pltpu.VMEM((RPW,), jnp.float32),
            pltpu.VMEM((NTP,), jnp.int32),
            pltpu.VMEM((GCH, HIDDEN), jnp.float32),
            pltpu.SemaphoreType.DMA,
        ],
    )(_scan_gather_body)
    return kern(x, mask2, scores2)


def _grouped_body(te_ref, xs_ref, wg_ref, wu_ref, wd_ref, ws_ref, out_ref):
    x = xs_ref[...]
    hg = jnp.dot(x, wg_ref[0], preferred_element_type=jnp.float32)
    hu = jnp.dot(x, wu_ref[0], preferred_element_type=jnp.float32)
    h = jax.nn.silu(hg) * hu
    o = jnp.dot(h, wd_ref[0], preferred_element_type=jnp.float32)
    out_ref[...] = o * ws_ref[...]


def _grouped(xs, ewg, ewu, ewd, wsort2d, te):
    grid_spec = pltpu.PrefetchScalarGridSpec(
        num_scalar_prefetch=1,
        grid=(NT,),
        in_specs=[
            pl.BlockSpec((B, HIDDEN), lambda i, te: (i, 0)),
            pl.BlockSpec((1, HIDDEN, INTER), lambda i, te: (te[i], 0, 0)),
            pl.BlockSpec((1, HIDDEN, INTER), lambda i, te: (te[i], 0, 0)),
            pl.BlockSpec((1, INTER, HIDDEN), lambda i, te: (te[i], 0, 0)),
            pl.BlockSpec((B, 1), lambda i, te: (i, 0)),
        ],
        out_specs=pl.BlockSpec((B, HIDDEN), lambda i, te: (i, 0)),
    )
    return pl.pallas_call(
        _grouped_body,
        grid_spec=grid_spec,
        out_shape=jax.ShapeDtypeStruct((NR, HIDDEN), jnp.float32),
        interpret=_INTERP,
    )(te, xs, ewg, ewu, ewd, wsort2d)


def _combine_body(outs_hbm, pos0_hbm, pos1_hbm, shared_hbm, out_hbm,
                  p0v, p1v, rows0, rows1, shbuf, obuf, sem):
    cid = lax.axis_index("c")
    sid = lax.axis_index("s")
    wid = cid * NS + sid
    t0 = wid * TPW

    pltpu.sync_copy(pos0_hbm.at[pl.ds(t0, TPW)], p0v)
    pltpu.sync_copy(pos1_hbm.at[pl.ds(t0, TPW)], p1v)
    for ch in range(TPW // 16):
        idx0 = p0v.at[pl.ds(ch * 16, 16)]
        idx1 = p1v.at[pl.ds(ch * 16, 16)]
        pltpu.async_copy(outs_hbm.at[idx0], rows0, sem).wait()
        pltpu.async_copy(outs_hbm.at[idx1], rows1, sem).wait()
        pltpu.sync_copy(shared_hbm.at[pl.ds(t0 + ch * 16, 16)], shbuf)

        def body(j, _):
            for i in range(16):
                sl = pl.ds(j * 16, 16)
                obuf[i, sl] = shbuf[i, sl] + rows0[i, sl] + rows1[i, sl]
            return 0

        lax.fori_loop(0, HIDDEN // 16, body, 0)
        pltpu.sync_copy(obuf, out_hbm.at[pl.ds(t0 + ch * 16, 16)])


def _combine(outs, pos0, pos1, shared_out):
    mesh = plsc.VectorSubcoreMesh(core_axis_name="c", subcore_axis_name="s")
    kern = functools.partial(
        pl.kernel,
        mesh=mesh,
        compiler_params=pltpu.CompilerParams(needs_layout_passes=False),
        out_type=jax.ShapeDtypeStruct((TOKENS, HIDDEN), jnp.float32),
        scratch_types=[
            pltpu.VMEM((TPW,), jnp.int32),
            pltpu.VMEM((TPW,), jnp.int32),
            pltpu.VMEM((16, HIDDEN), jnp.float32),
            pltpu.VMEM((16, HIDDEN), jnp.float32),
            pltpu.VMEM((16, HIDDEN), jnp.float32),
            pltpu.VMEM((16, HIDDEN), jnp.float32),
            pltpu.SemaphoreType.DMA,
        ],
    )(_combine_body)
    return kern(outs, pos0, pos1, shared_out)


def kernel(hidden_states, gate_weight, gate_bias, expert_w_gate, expert_w_up,
           expert_w_down, shared_w_gate, shared_w_up, shared_w_down):
    x = hidden_states.astype(jnp.float32)
    scores, mask = _routing(x, gate_weight, gate_bias)
    xs, wsort, pos0, pos1, te = _scan_gather(
        x, mask.reshape(-1), scores.reshape(-1))
    outs = _grouped(xs, expert_w_gate, expert_w_up, expert_w_down,
                    wsort.reshape(NR, 1), te)
    shared_out = _shared(x, shared_w_gate, shared_w_up, shared_w_down)
    return _combine(outs, pos0, pos1, shared_out)


# speedup vs baseline: 1.0612x; 1.0612x over previous
import functools

import jax
import jax.numpy as jnp
from jax import lax
from jax.experimental import pallas as pl
from jax.experimental.pallas import tpu as pltpu
from jax.experimental.pallas import tpu_sc as plsc

NUM_EXPERTS = 8
TOP_K = 2
HIDDEN = 1024
INTER = 512
N_GROUP = 4
GSZ = NUM_EXPERTS // N_GROUP
TOPK_GROUP = 2
ROUTED_SCALE = 2.5
TOKENS = 2048

TBLK = 256
B = 128
NT = 40
NTP = 48
NR = NT * B

NC = 2
NS = 16
NW = NC * NS
TPW = TOKENS // NW
TPS = TOKENS // NS
RPW = NR // NW
GCH = 40
CT = 8

_INTERP = False


def _routing_body(x_ref, gwt_ref, gb_ref, scores_ref, mask_ref):
    x = x_ref[...]
    logits = jnp.dot(x, gwt_ref[...], preferred_element_type=jnp.float32)
    s = jax.nn.sigmoid(logits)
    swb = s + gb_ref[...]

    col = lambda a, i: a[:, i : i + 1]
    g = [sum(col(swb, gi * GSZ + j) for j in range(GSZ)) for gi in range(N_GROUP)]
    gsel = []
    for gi in range(N_GROUP):
        rank = jnp.zeros_like(g[gi])
        for gj in range(N_GROUP):
            if gj == gi:
                continue
            beats = g[gj] > g[gi]
            if gj < gi:
                beats = beats | (g[gj] == g[gi])
            rank = rank + beats.astype(jnp.float32)
        gsel.append(rank < TOPK_GROUP)
    swbm = [jnp.where(gsel[e // GSZ], col(swb, e), 0.0) for e in range(NUM_EXPERTS)]
    sel = []
    for e in range(NUM_EXPERTS):
        rank = jnp.zeros_like(swbm[e])
        for e2 in range(NUM_EXPERTS):
            if e2 == e:
                continue
            beats = swbm[e2] > swbm[e]
            if e2 < e:
                beats = beats | (swbm[e2] == swbm[e])
            rank = rank + beats.astype(jnp.float32)
        sel.append(rank < TOP_K)
    sc = [jnp.where(sel[e], col(s, e), 0.0) for e in range(NUM_EXPERTS)]
    denom = sum(sc) + 1e-20
    w = [sc[e] / denom * ROUTED_SCALE for e in range(NUM_EXPERTS)]
    scores_ref[...] = jnp.concatenate(w, axis=1)
    mask_ref[...] = jnp.concatenate(
        [sel[e].astype(jnp.int32) for e in range(NUM_EXPERTS)], axis=1
    )


def _routing(x, gate_weight, gate_bias):
    nblk = TOKENS // TBLK
    return pl.pallas_call(
        _routing_body,
        grid=(nblk,),
        in_specs=[
            pl.BlockSpec((TBLK, HIDDEN), lambda t: (t, 0)),
            pl.BlockSpec((HIDDEN, NUM_EXPERTS), lambda t: (0, 0)),
            pl.BlockSpec((1, NUM_EXPERTS), lambda t: (0, 0)),
        ],
        out_specs=[
            pl.BlockSpec((TBLK, NUM_EXPERTS), lambda t: (t, 0)),
            pl.BlockSpec((TBLK, NUM_EXPERTS), lambda t: (t, 0)),
        ],
        out_shape=[
            jax.ShapeDtypeStruct((TOKENS, NUM_EXPERTS), jnp.float32),
            jax.ShapeDtypeStruct((TOKENS, NUM_EXPERTS), jnp.int32),
        ],
        interpret=_INTERP,
    )(x, gate_weight.T, gate_bias.reshape(1, NUM_EXPERTS))


def _shared_body(x_ref, wg_ref, wu_ref, wd_ref, out_ref):
    x = x_ref[...]
    hg = jnp.dot(x, wg_ref[...], preferred_element_type=jnp.float32)
    hu = jnp.dot(x, wu_ref[...], preferred_element_type=jnp.float32)
    h = jax.nn.silu(hg) * hu
    out_ref[...] = jnp.dot(h, wd_ref[...], preferred_element_type=jnp.float32)


def _shared(x, wg, wu, wd):
    nblk = TOKENS // TBLK
    return pl.pallas_call(
        _shared_body,
        grid=(nblk,),
        in_specs=[
            pl.BlockSpec((TBLK, HIDDEN), lambda t: (t, 0)),
            pl.BlockSpec((HIDDEN, INTER), lambda t: (0, 0)),
            pl.BlockSpec((HIDDEN, INTER), lambda t: (0, 0)),
            pl.BlockSpec((INTER, HIDDEN), lambda t: (0, 0)),
        ],
        out_specs=pl.BlockSpec((TBLK, HIDDEN), lambda t: (t, 0)),
        out_shape=jax.ShapeDtypeStruct((TOKENS, HIDDEN), jnp.float32),
        interpret=_INTERP,
    )(x, wg, wu, wd)


def _iota16():
    return lax.iota(jnp.int32, 16)


def _splat_i(s):
    return jnp.full((16,), s, jnp.int32)


def _scan_gather_body(x_hbm, mask_hbm, scores_hbm,
                      xs_hbm, wsort_hbm, pos0_hbm, pos1_hbm, te_hbm,
                      mvm, svm, cntv, partials_v, partials_sh,
                      p0loc, p1loc, w0loc, w1loc,
                      p0sh, p1sh, w0sh, w1sh,
                      p0all, p1all, w0all, w1all,
                      stloc, wsloc, teloc, rowbuf, sem, wsem):
    cid = lax.axis_index("c")
    sid = lax.axis_index("s")
    wid = cid * NS + sid
    t0 = sid * TPS

    pltpu.sync_copy(mask_hbm.at[pl.ds(t0 * NUM_EXPERTS, TPS * NUM_EXPERTS)], mvm)
    pltpu.sync_copy(scores_hbm.at[pl.ds(t0 * NUM_EXPERTS, TPS * NUM_EXPERTS)], svm)

    it16 = _iota16()
    cnt_vec = jnp.zeros((16,), jnp.int32)
    for e in range(NUM_EXPERTS):
        acc = jnp.zeros((16,), jnp.int32)
        for ch in range(TPS // 16):
            fidx = (it16 + (ch * 16)) * NUM_EXPERTS + e
            acc = acc + plsc.load_gather(mvm, [fidx])
        cnt_e = jnp.sum(acc)
        cnt_vec = cnt_vec + jnp.where(it16 == e, cnt_e, 0)
    cntv[...] = cnt_vec
    pltpu.sync_copy(cntv, partials_sh.at[sid])
    plsc.subcore_barrier()

    pltpu.sync_copy(partials_sh, partials_v)
    base_vec = jnp.zeros((16,), jnp.int32)
    totc_vec = jnp.zeros((16,), jnp.int32)
    for s2 in range(NS):
        row = partials_v[s2]
        pred = jnp.full((16,), s2 < sid)
        base_vec = base_vec + jnp.where(pred, row, 0)
        totc_vec = totc_vec + row
    tiles_vec = (totc_vec + (B - 1)) // B
    cumt_vec = jnp.cumsum(tiles_vec)
    seg_vec = (cumt_vec - tiles_vec) * B

    carry = [seg_vec[e] + base_vec[e] for e in range(NUM_EXPERTS)]
    for ch in range(TPS // 16):
        tokidx = it16 + (ch * 16)
        acc_k = jnp.zeros((16,), jnp.int32)
        p0v = jnp.zeros((16,), jnp.int32)
        p1v = jnp.zeros((16,), jnp.int32)
        w0v = jnp.zeros((16,), jnp.float32)
        w1v = jnp.zeros((16,), jnp.float32)
        for e in range(NUM_EXPERTS):
            fidx = tokidx * NUM_EXPERTS + e
            mv = plsc.load_gather(mvm, [fidx])
            sv = plsc.load_gather(svm, [fidx])
            excl = jnp.cumsum(mv) - mv
            posv = excl + carry[e]
            selb = mv > 0
            first = selb & (acc_k == 0)
            second = selb & (acc_k == 1)
            p0v = jnp.where(first, posv, p0v)
            w0v = jnp.where(first, sv, w0v)
            p1v = jnp.where(second, posv, p1v)
            w1v = jnp.where(second, sv, w1v)
            acc_k = acc_k + mv
            carry[e] = carry[e] + jnp.sum(mv)
        p0loc[pl.ds(ch * 16, 16)] = p0v
        p1loc[pl.ds(ch * 16, 16)] = p1v
        w0loc[pl.ds(ch * 16, 16)] = w0v
        w1loc[pl.ds(ch * 16, 16)] = w1v

    pltpu.sync_copy(p0loc, p0sh.at[pl.ds(t0, TPS)])
    pltpu.sync_copy(p1loc, p1sh.at[pl.ds(t0, TPS)])
    pltpu.sync_copy(w0loc, w0sh.at[pl.ds(t0, TPS)])
    pltpu.sync_copy(w1loc, w1sh.at[pl.ds(t0, TPS)])

    @pl.when(cid == 0)
    def _():
        pltpu.sync_copy(p0loc, pos0_hbm.at[pl.ds(t0, TPS)])

    @pl.when(cid == 1)
    def _():
        pltpu.sync_copy(p1loc, pos1_hbm.at[pl.ds(t0, TPS)])

    @pl.when(wid == 0)
    def _():
        for ch in range(NTP // 16):
            jv = it16 + (ch * 16)
            te = jnp.zeros((16,), jnp.int32)
            for e in range(NUM_EXPERTS):
                te = te + (jv >= cumt_vec[e]).astype(jnp.int32)
            teloc[pl.ds(ch * 16, 16)] = jnp.minimum(te, NUM_EXPERTS - 1)
        pltpu.sync_copy(teloc, te_hbm)

    plsc.subcore_barrier()

    pltpu.sync_copy(p0sh, p0all)
    pltpu.sync_copy(p1sh, p1all)
    pltpu.sync_copy(w0sh, w0all)
    pltpu.sync_copy(w1sh, w1all)

    r0 = wid * RPW
    zi = jnp.zeros((16,), jnp.int32)
    zf = jnp.zeros((16,), jnp.float32)
    for ch in range(RPW // 16):
        stloc[pl.ds(ch * 16, 16)] = zi
        wsloc[pl.ds(ch * 16, 16)] = zf
    for ch in range(TOKENS // 16):
        sl = pl.ds(ch * 16, 16)
        tokv = it16 + (ch * 16)
        for pall, wall in ((p0all, w0all), (p1all, w1all)):
            pv = pall[sl] - r0
            wv = wall[sl]
            inb = (pv >= 0) & (pv < RPW)
            plsc.store_scatter(stloc, [pv], tokv, mask=inb)
            plsc.store_scatter(wsloc, [pv], wv, mask=inb)

    pltpu.sync_copy(wsloc, wsort_hbm.at[pl.ds(r0, RPW)])

    nch = RPW // GCH
    gats = {}
    wbs = {}

    def _start(ch, buf_i):
        idx = stloc.at[pl.ds(ch * GCH, GCH)]
        gats[buf_i] = pltpu.async_copy(x_hbm.at[idx], rowbuf.at[buf_i],
                                       sem.at[buf_i])

    _start(0, 0)
    for ch in range(nch):
        b = ch % 2
        nb = 1 - b
        if ch + 1 < nch:
            if nb in wbs:
                wbs[nb].wait()
            _start(ch + 1, nb)
        gats[b].wait()
        wbs[b] = pltpu.async_copy(rowbuf.at[b],
                                  xs_hbm.at[pl.ds(r0 + ch * GCH, GCH)],
                                  wsem.at[b])
    for b in wbs:
        wbs[b].wait()


def _scan_gather(x, mask2, scores2):
    mesh = plsc.VectorSubcoreMesh(core_axis_name="c", subcore_axis_name="s")
    kern = functools.partial(
        pl.kernel,
        mesh=mesh,
        compiler_params=pltpu.CompilerParams(needs_layout_passes=False),
        out_type=[
            jax.ShapeDtypeStruct((NR, HIDDEN), jnp.float32),
            jax.ShapeDtypeStruct((NR,), jnp.float32),
            jax.ShapeDtypeStruct((TOKENS,), jnp.int32),
            jax.ShapeDtypeStruct((TOKENS,), jnp.int32),
            jax.ShapeDtypeStruct((NTP,), jnp.int32),
        ],
        scratch_types=[
            pltpu.VMEM((TPS * NUM_EXPERTS,), jnp.int32),
            pltpu.VMEM((TPS * NUM_EXPERTS,), jnp.float32),
            pltpu.VMEM((16,), jnp.int32),
            pltpu.VMEM((NS, 16), jnp.int32),
            pltpu.VMEM_SHARED((NS, 16), jnp.int32),
            pltpu.VMEM((TPS,), jnp.int32),
            pltpu.VMEM((TPS,), jnp.int32),
            pltpu.VMEM((TPS,), jnp.float32),
            pltpu.VMEM((TPS,), jnp.float32),
            pltpu.VMEM_SHARED((TOKENS,), jnp.int32),
            pltpu.VMEM_SHARED((TOKENS,), jnp.int32),
            pltpu.VMEM_SHARED((TOKENS,), jnp.float32),
            pltpu.VMEM_SHARED((TOKENS,), jnp.float32),
            pltpu.VMEM((TOKENS,), jnp.int32),
            pltpu.VMEM((TOKENS,), jnp.int32),
            pltpu.VMEM((TOKENS,), jnp.float32),
            pltpu.VMEM((TOKENS,), jnp.float32),
            pltpu.VMEM((RPW,), jnp.int32),
            pltpu.VMEM((RPW,), jnp.float32),
            pltpu.VMEM((NTP,), jnp.int32),
            pltpu.VMEM((2, GCH, HIDDEN), jnp.float32),
            pltpu.SemaphoreType.DMA((2,)),
            pltpu.SemaphoreType.DMA((2,)),
        ],
    )(_scan_gather_body)
    return kern(x, mask2, scores2)


def _grouped_body(te_ref, xs_ref, wg_ref, wu_ref, wd_ref, ws_ref, out_ref):
    x = xs_ref[...]
    hg = jnp.dot(x, wg_ref[0], preferred_element_type=jnp.float32)
    hu = jnp.dot(x, wu_ref[0], preferred_element_type=jnp.float32)
    h = jax.nn.silu(hg) * hu
    o = jnp.dot(h, wd_ref[0], preferred_element_type=jnp.float32)
    out_ref[...] = o * ws_ref[...]


def _grouped(xs, ewg, ewu, ewd, wsort2d, te):
    grid_spec = pltpu.PrefetchScalarGridSpec(
        num_scalar_prefetch=1,
        grid=(NT,),
        in_specs=[
            pl.BlockSpec((B, HIDDEN), lambda i, te: (i, 0)),
            pl.BlockSpec((1, HIDDEN, INTER), lambda i, te: (te[i], 0, 0)),
            pl.BlockSpec((1, HIDDEN, INTER), lambda i, te: (te[i], 0, 0)),
            pl.BlockSpec((1, INTER, HIDDEN), lambda i, te: (te[i], 0, 0)),
            pl.BlockSpec((B, 1), lambda i, te: (i, 0)),
        ],
        out_specs=pl.BlockSpec((B, HIDDEN), lambda i, te: (i, 0)),
    )
    return pl.pallas_call(
        _grouped_body,
        grid_spec=grid_spec,
        out_shape=jax.ShapeDtypeStruct((NR, HIDDEN), jnp.float32),
        interpret=_INTERP,
    )(te, xs, ewg, ewu, ewd, wsort2d)


def _combine_body(outs_hbm, pos0_hbm, pos1_hbm, shared_hbm, out_hbm,
                  p0v, p1v, rows0, rows1, shbuf, obuf, gsem, wsem):
    cid = lax.axis_index("c")
    sid = lax.axis_index("s")
    wid = cid * NS + sid
    t0 = wid * TPW
    nch = TPW // CT
    nsl = HIDDEN // 16

    pltpu.sync_copy(pos0_hbm.at[pl.ds(t0, TPW)], p0v)
    pltpu.sync_copy(pos1_hbm.at[pl.ds(t0, TPW)], p1v)

    gats = {}
    wbs = {}

    def _start(ch, b):
        idx0 = p0v.at[pl.ds(ch * CT, CT)]
        idx1 = p1v.at[pl.ds(ch * CT, CT)]
        gats[b] = (
            pltpu.async_copy(outs_hbm.at[idx0], rows0.at[b], gsem.at[b]),
            pltpu.async_copy(outs_hbm.at[idx1], rows1.at[b], gsem.at[b]),
            pltpu.async_copy(shared_hbm.at[pl.ds(t0 + ch * CT, CT)],
                             shbuf.at[b], gsem.at[b]),
        )

    _start(0, 0)
    for ch in range(nch):
        b = ch % 2
        nb = 1 - b
        if ch + 1 < nch:
            if nb in wbs:
                wbs[nb].wait()
            _start(ch + 1, nb)
        for cp in gats[b]:
            cp.wait()
        ob = obuf.at[b]
        r0b = rows0.at[b]
        r1b = rows1.at[b]
        shb = shbuf.at[b]

        @plsc.parallel_loop(0, CT * nsl, 1, unroll=8)
        def _(j):
            i = j // nsl
            sl = pl.ds((j % nsl) * 16, 16)
            ob[i, sl] = shb[i, sl] + r0b[i, sl] + r1b[i, sl]

        wbs[b] = pltpu.async_copy(obuf.at[b],
                                  out_hbm.at[pl.ds(t0 + ch * CT, CT)],
                                  wsem.at[b])
    for b in wbs:
        wbs[b].wait()


def _combine(outs, pos0, pos1, shared_out):
    mesh = plsc.VectorSubcoreMesh(core_axis_name="c", subcore_axis_name="s")
    kern = functools.partial(
        pl.kernel,
        mesh=mesh,
        compiler_params=pltpu.CompilerParams(needs_layout_passes=False),
        out_type=jax.ShapeDtypeStruct((TOKENS, HIDDEN), jnp.float32),
        scratch_types=[
            pltpu.VMEM((TPW,), jnp.int32),
            pltpu.VMEM((TPW,), jnp.int32),
            pltpu.VMEM((2, CT, HIDDEN), jnp.float32),
            pltpu.VMEM((2, CT, HIDDEN), jnp.float32),
            pltpu.VMEM((2, CT, HIDDEN), jnp.float32),
            pltpu.VMEM((2, CT, HIDDEN), jnp.float32),
            pltpu.SemaphoreType.DMA((2,)),
            pltpu.SemaphoreType.DMA((2,)),
        ],
    )(_combine_body)
    return kern(outs, pos0, pos1, shared_out)


def kernel(hidden_states, gate_weight, gate_bias, expert_w_gate, expert_w_up,
           expert_w_down, shared_w_gate, shared_w_up, shared_w_down):
    x = hidden_states.astype(jnp.float32)
    scores, mask = _routing(x, gate_weight, gate_bias)
    xs, wsort, pos0, pos1, te = _scan_gather(
        x, mask.reshape(-1), scores.reshape(-1))
    outs = _grouped(xs, expert_w_gate, expert_w_up, expert_w_down,
                    wsort.reshape(NR, 1), te)
    shared_out = _shared(x, shared_w_gate, shared_w_up, shared_w_down)
    return _combine(outs, pos0, pos1, shared_out)


# speedup vs baseline: 1.4887x; 1.4029x over previous
import functools

import jax
import jax.numpy as jnp
from jax import lax
from jax.experimental import pallas as pl
from jax.experimental.pallas import tpu as pltpu
from jax.experimental.pallas import tpu_sc as plsc

NUM_EXPERTS = 8
TOP_K = 2
HIDDEN = 1024
INTER = 512
N_GROUP = 4
GSZ = NUM_EXPERTS // N_GROUP
TOPK_GROUP = 2
ROUTED_SCALE = 2.5
TOKENS = 2048

TBLK = 256
B = 128
NT = 40
NTP = 48
NR = NT * B

NC = 2
NS = 16
NW = NC * NS
TPW = TOKENS // NW
TPS = TOKENS // NS
RPW = NR // NW
GCH = 16
NBUF = 4
CT = 8

_INTERP = False


def _routing_body(x_ref, gwt_ref, gb_ref, scores_ref, mask_ref):
    x = x_ref[...]
    logits = jnp.dot(x, gwt_ref[...], preferred_element_type=jnp.float32)
    s = jax.nn.sigmoid(logits)
    swb = s + gb_ref[...]

    col = lambda a, i: a[:, i : i + 1]
    g = [sum(col(swb, gi * GSZ + j) for j in range(GSZ)) for gi in range(N_GROUP)]
    gsel = []
    for gi in range(N_GROUP):
        rank = jnp.zeros_like(g[gi])
        for gj in range(N_GROUP):
            if gj == gi:
                continue
            beats = g[gj] > g[gi]
            if gj < gi:
                beats = beats | (g[gj] == g[gi])
            rank = rank + beats.astype(jnp.float32)
        gsel.append(rank < TOPK_GROUP)
    swbm = [jnp.where(gsel[e // GSZ], col(swb, e), 0.0) for e in range(NUM_EXPERTS)]
    sel = []
    for e in range(NUM_EXPERTS):
        rank = jnp.zeros_like(swbm[e])
        for e2 in range(NUM_EXPERTS):
            if e2 == e:
                continue
            beats = swbm[e2] > swbm[e]
            if e2 < e:
                beats = beats | (swbm[e2] == swbm[e])
            rank = rank + beats.astype(jnp.float32)
        sel.append(rank < TOP_K)
    sc = [jnp.where(sel[e], col(s, e), 0.0) for e in range(NUM_EXPERTS)]
    denom = sum(sc) + 1e-20
    w = [sc[e] / denom * ROUTED_SCALE for e in range(NUM_EXPERTS)]
    scores_ref[...] = jnp.concatenate(w, axis=1)
    mask_ref[...] = jnp.concatenate(
        [sel[e].astype(jnp.int32) for e in range(NUM_EXPERTS)], axis=1
    )


def _routing(x, gate_weight, gate_bias):
    nblk = TOKENS // TBLK
    return pl.pallas_call(
        _routing_body,
        grid=(nblk,),
        in_specs=[
            pl.BlockSpec((TBLK, HIDDEN), lambda t: (t, 0)),
            pl.BlockSpec((HIDDEN, NUM_EXPERTS), lambda t: (0, 0)),
            pl.BlockSpec((1, NUM_EXPERTS), lambda t: (0, 0)),
        ],
        out_specs=[
            pl.BlockSpec((TBLK, NUM_EXPERTS), lambda t: (t, 0)),
            pl.BlockSpec((TBLK, NUM_EXPERTS), lambda t: (t, 0)),
        ],
        out_shape=[
            jax.ShapeDtypeStruct((TOKENS, NUM_EXPERTS), jnp.float32),
            jax.ShapeDtypeStruct((TOKENS, NUM_EXPERTS), jnp.int32),
        ],
        interpret=_INTERP,
    )(x, gate_weight.T, gate_bias.reshape(1, NUM_EXPERTS))


def _shared_body(x_ref, wg_ref, wu_ref, wd_ref, out_ref):
    x = x_ref[...]
    hg = jnp.dot(x, wg_ref[...], preferred_element_type=jnp.float32)
    hu = jnp.dot(x, wu_ref[...], preferred_element_type=jnp.float32)
    h = jax.nn.silu(hg) * hu
    out_ref[...] = jnp.dot(h, wd_ref[...], preferred_element_type=jnp.float32)


def _shared(x, wg, wu, wd):
    nblk = TOKENS // TBLK
    return pl.pallas_call(
        _shared_body,
        grid=(nblk,),
        in_specs=[
            pl.BlockSpec((TBLK, HIDDEN), lambda t: (t, 0)),
            pl.BlockSpec((HIDDEN, INTER), lambda t: (0, 0)),
            pl.BlockSpec((HIDDEN, INTER), lambda t: (0, 0)),
            pl.BlockSpec((INTER, HIDDEN), lambda t: (0, 0)),
        ],
        out_specs=pl.BlockSpec((TBLK, HIDDEN), lambda t: (t, 0)),
        out_shape=jax.ShapeDtypeStruct((TOKENS, HIDDEN), jnp.float32),
        interpret=_INTERP,
    )(x, wg, wu, wd)


def _iota16():
    return lax.iota(jnp.int32, 16)


def _splat_i(s):
    return jnp.full((16,), s, jnp.int32)


def _scan_gather_body(x_hbm, mask_hbm, scores_hbm,
                      xs_hbm, wsort_hbm, pos0_hbm, pos1_hbm, te_hbm,
                      mvm, svm, cntv, partials_v, partials_sh,
                      p0loc, p1loc, w0loc, w1loc,
                      p0sh, p1sh, w0sh, w1sh,
                      p0all, p1all, w0all, w1all,
                      stloc, wsloc, teloc, rowbuf, sem, wsem):
    cid = lax.axis_index("c")
    sid = lax.axis_index("s")
    wid = cid * NS + sid
    t0 = sid * TPS

    scope = jax.named_scope
    pltpu.sync_copy(mask_hbm.at[pl.ds(t0 * NUM_EXPERTS, TPS * NUM_EXPERTS)], mvm)
    pltpu.sync_copy(scores_hbm.at[pl.ds(t0 * NUM_EXPERTS, TPS * NUM_EXPERTS)], svm)

    sA = scope("scanA"); sA.__enter__()
    it16 = _iota16()
    cnt_vec = jnp.zeros((16,), jnp.int32)
    for e in range(NUM_EXPERTS):
        acc = jnp.zeros((16,), jnp.int32)
        for ch in range(TPS // 16):
            fidx = (it16 + (ch * 16)) * NUM_EXPERTS + e
            acc = acc + plsc.load_gather(mvm, [fidx])
        cnt_e = jnp.sum(acc)
        cnt_vec = cnt_vec + jnp.where(it16 == e, cnt_e, 0)
    cntv[...] = cnt_vec
    pltpu.sync_copy(cntv, partials_sh.at[sid])
    plsc.subcore_barrier()

    pltpu.sync_copy(partials_sh, partials_v)
    base_vec = jnp.zeros((16,), jnp.int32)
    totc_vec = jnp.zeros((16,), jnp.int32)
    for s2 in range(NS):
        row = partials_v[s2]
        pred = jnp.full((16,), s2 < sid)
        base_vec = base_vec + jnp.where(pred, row, 0)
        totc_vec = totc_vec + row
    tiles_vec = (totc_vec + (B - 1)) // B
    cumt_vec = jnp.cumsum(tiles_vec)
    seg_vec = (cumt_vec - tiles_vec) * B

    sA.__exit__(None, None, None)
    sB = scope("scanB"); sB.__enter__()
    carry = [seg_vec[e] + base_vec[e] for e in range(NUM_EXPERTS)]
    for ch in range(TPS // 16):
        tokidx = it16 + (ch * 16)
        acc_k = jnp.zeros((16,), jnp.int32)
        p0v = jnp.zeros((16,), jnp.int32)
        p1v = jnp.zeros((16,), jnp.int32)
        w0v = jnp.zeros((16,), jnp.float32)
        w1v = jnp.zeros((16,), jnp.float32)
        for e in range(NUM_EXPERTS):
            fidx = tokidx * NUM_EXPERTS + e
            mv = plsc.load_gather(mvm, [fidx])
            sv = plsc.load_gather(svm, [fidx])
            excl = jnp.cumsum(mv) - mv
            posv = excl + carry[e]
            selb = mv > 0
            first = selb & (acc_k == 0)
            second = selb & (acc_k == 1)
            p0v = jnp.where(first, posv, p0v)
            w0v = jnp.where(first, sv, w0v)
            p1v = jnp.where(second, posv, p1v)
            w1v = jnp.where(second, sv, w1v)
            acc_k = acc_k + mv
            carry[e] = carry[e] + jnp.sum(mv)
        p0loc[pl.ds(ch * 16, 16)] = p0v
        p1loc[pl.ds(ch * 16, 16)] = p1v
        w0loc[pl.ds(ch * 16, 16)] = w0v
        w1loc[pl.ds(ch * 16, 16)] = w1v

    pltpu.sync_copy(p0loc, p0sh.at[pl.ds(t0, TPS)])
    pltpu.sync_copy(p1loc, p1sh.at[pl.ds(t0, TPS)])
    pltpu.sync_copy(w0loc, w0sh.at[pl.ds(t0, TPS)])
    pltpu.sync_copy(w1loc, w1sh.at[pl.ds(t0, TPS)])

    @pl.when(cid == 0)
    def _():
        pltpu.sync_copy(p0loc, pos0_hbm.at[pl.ds(t0, TPS)])

    @pl.when(cid == 1)
    def _():
        pltpu.sync_copy(p1loc, pos1_hbm.at[pl.ds(t0, TPS)])

    @pl.when(wid == 0)
    def _():
        for ch in range(NTP // 16):
            jv = it16 + (ch * 16)
            te = jnp.zeros((16,), jnp.int32)
            for e in range(NUM_EXPERTS):
                te = te + (jv >= cumt_vec[e]).astype(jnp.int32)
            teloc[pl.ds(ch * 16, 16)] = jnp.minimum(te, NUM_EXPERTS - 1)
        pltpu.sync_copy(teloc, te_hbm)

    plsc.subcore_barrier()

    sB.__exit__(None, None, None)
    sC = scope("scanC"); sC.__enter__()
    pltpu.sync_copy(p0sh, p0all)
    pltpu.sync_copy(p1sh, p1all)
    pltpu.sync_copy(w0sh, w0all)
    pltpu.sync_copy(w1sh, w1all)

    r0 = wid * RPW
    zf = jnp.zeros((16,), jnp.float32)
    for ch in range(RPW // 16):
        stloc[pl.ds(ch * 16, 16)] = (it16 + (r0 + ch * 16)) & (TOKENS - 1)
        wsloc[pl.ds(ch * 16, 16)] = zf
    for ch in range(TOKENS // 16):
        sl = pl.ds(ch * 16, 16)
        tokv = it16 + (ch * 16)
        for pall, wall in ((p0all, w0all), (p1all, w1all)):
            pv = pall[sl] - r0
            wv = wall[sl]
            inb = (pv >= 0) & (pv < RPW)
            plsc.store_scatter(stloc, [pv], tokv, mask=inb)
            plsc.store_scatter(wsloc, [pv], wv, mask=inb)

    pltpu.sync_copy(wsloc, wsort_hbm.at[pl.ds(r0, RPW)])

    sC.__exit__(None, None, None)
    sD = scope("scanD"); sD.__enter__()
    nch = RPW // GCH
    gats = {}
    wbs = {}

    def _start(ch):
        buf_i = ch % NBUF
        idx = stloc.at[pl.ds(ch * GCH, GCH)]
        gats[buf_i] = pltpu.async_copy(x_hbm.at[idx], rowbuf.at[buf_i],
                                       sem.at[buf_i])

    for ch in range(min(NBUF - 1, nch)):
        _start(ch)
    for ch in range(nch):
        b = ch % NBUF
        if ch + NBUF - 1 < nch:
            nb = (ch + NBUF - 1) % NBUF
            if nb in wbs:
                wbs[nb].wait()
            _start(ch + NBUF - 1)
        gats[b].wait()
        _UNUSED = pltpu.async_copy(rowbuf.at[b],
                                  xs_hbm.at[pl.ds(r0 + ch * GCH, GCH)],
                                  wsem.at[b])
        wbs[b] = _UNUSED
    for b in wbs:
        wbs[b].wait()
    sD.__exit__(None, None, None)


def _scan_gather(x, mask2, scores2):
    mesh = plsc.VectorSubcoreMesh(core_axis_name="c", subcore_axis_name="s")
    kern = functools.partial(
        pl.kernel,
        mesh=mesh,
        compiler_params=pltpu.CompilerParams(needs_layout_passes=False),
        out_type=[
            jax.ShapeDtypeStruct((NR, HIDDEN), jnp.float32),
            jax.ShapeDtypeStruct((NR,), jnp.float32),
            jax.ShapeDtypeStruct((TOKENS,), jnp.int32),
            jax.ShapeDtypeStruct((TOKENS,), jnp.int32),
            jax.ShapeDtypeStruct((NTP,), jnp.int32),
        ],
        scratch_types=[
            pltpu.VMEM((TPS * NUM_EXPERTS,), jnp.int32),
            pltpu.VMEM((TPS * NUM_EXPERTS,), jnp.float32),
            pltpu.VMEM((16,), jnp.int32),
            pltpu.VMEM((NS, 16), jnp.int32),
            pltpu.VMEM_SHARED((NS, 16), jnp.int32),
            pltpu.VMEM((TPS,), jnp.int32),
            pltpu.VMEM((TPS,), jnp.int32),
            pltpu.VMEM((TPS,), jnp.float32),
            pltpu.VMEM((TPS,), jnp.float32),
            pltpu.VMEM_SHARED((TOKENS,), jnp.int32),
            pltpu.VMEM_SHARED((TOKENS,), jnp.int32),
            pltpu.VMEM_SHARED((TOKENS,), jnp.float32),
            pltpu.VMEM_SHARED((TOKENS,), jnp.float32),
            pltpu.VMEM((TOKENS,), jnp.int32),
            pltpu.VMEM((TOKENS,), jnp.int32),
            pltpu.VMEM((TOKENS,), jnp.float32),
            pltpu.VMEM((TOKENS,), jnp.float32),
            pltpu.VMEM((RPW,), jnp.int32),
            pltpu.VMEM((RPW,), jnp.float32),
            pltpu.VMEM((NTP,), jnp.int32),
            pltpu.VMEM((NBUF, GCH, HIDDEN), jnp.float32),
            pltpu.SemaphoreType.DMA((NBUF,)),
            pltpu.SemaphoreType.DMA((NBUF,)),
        ],
    )(_scan_gather_body)
    return kern(x, mask2, scores2)


def _grouped_body(te_ref, xs_ref, wg_ref, wu_ref, wd_ref, ws_ref, out_ref):
    x = xs_ref[...]
    hg = jnp.dot(x, wg_ref[0], preferred_element_type=jnp.float32)
    hu = jnp.dot(x, wu_ref[0], preferred_element_type=jnp.float32)
    h = jax.nn.silu(hg) * hu
    o = jnp.dot(h, wd_ref[0], preferred_element_type=jnp.float32)
    out_ref[...] = o * ws_ref[...]


def _grouped(xs, ewg, ewu, ewd, wsort2d, te):
    grid_spec = pltpu.PrefetchScalarGridSpec(
        num_scalar_prefetch=1,
        grid=(NT,),
        in_specs=[
            pl.BlockSpec((B, HIDDEN), lambda i, te: (i, 0)),
            pl.BlockSpec((1, HIDDEN, INTER), lambda i, te: (te[i], 0, 0)),
            pl.BlockSpec((1, HIDDEN, INTER), lambda i, te: (te[i], 0, 0)),
            pl.BlockSpec((1, INTER, HIDDEN), lambda i, te: (te[i], 0, 0)),
            pl.BlockSpec((B, 1), lambda i, te: (i, 0)),
        ],
        out_specs=pl.BlockSpec((B, HIDDEN), lambda i, te: (i, 0)),
    )
    return pl.pallas_call(
        _grouped_body,
        grid_spec=grid_spec,
        out_shape=jax.ShapeDtypeStruct((NR, HIDDEN), jnp.float32),
        interpret=_INTERP,
    )(te, xs, ewg, ewu, ewd, wsort2d)


def _combine_body(outs_hbm, pos0_hbm, pos1_hbm, shared_hbm, out_hbm,
                  p0v, p1v, rows0, rows1, shbuf, obuf, gsem, wsem):
    cid = lax.axis_index("c")
    sid = lax.axis_index("s")
    wid = cid * NS + sid
    t0 = wid * TPW
    nch = TPW // CT
    nsl = HIDDEN // 16

    pltpu.sync_copy(pos0_hbm.at[pl.ds(t0, TPW)], p0v)
    pltpu.sync_copy(pos1_hbm.at[pl.ds(t0, TPW)], p1v)

    gats = {}
    wbs = {}

    def _start(ch, b):
        idx0 = p0v.at[pl.ds(ch * CT, CT)]
        idx1 = p1v.at[pl.ds(ch * CT, CT)]
        gats[b] = (
            pltpu.async_copy(outs_hbm.at[idx0], rows0.at[b], gsem.at[b]),
            pltpu.async_copy(outs_hbm.at[idx1], rows1.at[b], gsem.at[b]),
            pltpu.async_copy(shared_hbm.at[pl.ds(t0 + ch * CT, CT)],
                             shbuf.at[b], gsem.at[b]),
        )

    _start(0, 0)
    for ch in range(nch):
        b = ch % 2
        nb = 1 - b
        if ch + 1 < nch:
            if nb in wbs:
                wbs[nb].wait()
            _start(ch + 1, nb)
        for cp in gats[b]:
            cp.wait()
        ob = obuf.at[b]
        r0b = rows0.at[b]
        r1b = rows1.at[b]
        shb = shbuf.at[b]

        @plsc.parallel_loop(0, CT * nsl, 1, unroll=8)
        def _(j):
            i = j // nsl
            sl = pl.ds((j % nsl) * 16, 16)
            ob[i, sl] = shb[i, sl] + r0b[i, sl] + r1b[i, sl]

        wbs[b] = pltpu.async_copy(obuf.at[b],
                                  out_hbm.at[pl.ds(t0 + ch * CT, CT)],
                                  wsem.at[b])
    for b in wbs:
        wbs[b].wait()


def _combine(outs, pos0, pos1, shared_out):
    mesh = plsc.VectorSubcoreMesh(core_axis_name="c", subcore_axis_name="s")
    kern = functools.partial(
        pl.kernel,
        mesh=mesh,
        compiler_params=pltpu.CompilerParams(needs_layout_passes=False),
        out_type=jax.ShapeDtypeStruct((TOKENS, HIDDEN), jnp.float32),
        scratch_types=[
            pltpu.VMEM((TPW,), jnp.int32),
            pltpu.VMEM((TPW,), jnp.int32),
            pltpu.VMEM((2, CT, HIDDEN), jnp.float32),
            pltpu.VMEM((2, CT, HIDDEN), jnp.float32),
            pltpu.VMEM((2, CT, HIDDEN), jnp.float32),
            pltpu.VMEM((2, CT, HIDDEN), jnp.float32),
            pltpu.SemaphoreType.DMA((2,)),
            pltpu.SemaphoreType.DMA((2,)),
        ],
    )(_combine_body)
    return kern(outs, pos0, pos1, shared_out)


def kernel(hidden_states, gate_weight, gate_bias, expert_w_gate, expert_w_up,
           expert_w_down, shared_w_gate, shared_w_up, shared_w_down):
    x = hidden_states.astype(jnp.float32)
    scores, mask = _routing(x, gate_weight, gate_bias)
    xs, wsort, pos0, pos1, te = _scan_gather(
        x, mask.reshape(-1), scores.reshape(-1))
    outs = _grouped(xs, expert_w_gate, expert_w_up, expert_w_down,
                    wsort.reshape(NR, 1), te)
    shared_out = _shared(x, shared_w_gate, shared_w_up, shared_w_down)
    return _combine(outs, pos0, pos1, shared_out)


# speedup vs baseline: 1.7704x; 1.1892x over previous
import functools

import jax
import jax.numpy as jnp
from jax import lax
from jax.experimental import pallas as pl
from jax.experimental.pallas import tpu as pltpu
from jax.experimental.pallas import tpu_sc as plsc

NUM_EXPERTS = 8
TOP_K = 2
HIDDEN = 1024
INTER = 512
N_GROUP = 4
GSZ = NUM_EXPERTS // N_GROUP
TOPK_GROUP = 2
ROUTED_SCALE = 2.5
TOKENS = 2048

TBLK = 256
B = 128
NT = 40
NTP = 48
NR = NT * B

NC = 2
NS = 16
NW = NC * NS
TPW = TOKENS // NW
TPS = TOKENS // NS
RPW = NR // NW
GCH = 16
NBUF = 4
CT = 8

_INTERP = False


def _routing_body(x_ref, gw_ref, gb_ref, scores_ref, mask_ref):
    logits = lax.dot_general(gw_ref[...], x_ref[...],
                             dimension_numbers=(((1,), (1,)), ((), ())),
                             preferred_element_type=jnp.float32)
    s = jax.nn.sigmoid(logits)
    swb = s + gb_ref[...]

    row = lambda a, i: a[i : i + 1, :]
    g = [sum(row(swb, gi * GSZ + j) for j in range(GSZ)) for gi in range(N_GROUP)]
    gsel = []
    for gi in range(N_GROUP):
        rank = jnp.zeros_like(g[gi])
        for gj in range(N_GROUP):
            if gj == gi:
                continue
            beats = g[gj] > g[gi]
            if gj < gi:
                beats = beats | (g[gj] == g[gi])
            rank = rank + beats.astype(jnp.float32)
        gsel.append(rank < TOPK_GROUP)
    swbm = [jnp.where(gsel[e // GSZ], row(swb, e), 0.0) for e in range(NUM_EXPERTS)]
    sel = []
    for e in range(NUM_EXPERTS):
        rank = jnp.zeros_like(swbm[e])
        for e2 in range(NUM_EXPERTS):
            if e2 == e:
                continue
            beats = swbm[e2] > swbm[e]
            if e2 < e:
                beats = beats | (swbm[e2] == swbm[e])
            rank = rank + beats.astype(jnp.float32)
        sel.append(rank < TOP_K)
    sc = [jnp.where(sel[e], row(s, e), 0.0) for e in range(NUM_EXPERTS)]
    denom = sum(sc) + 1e-20
    w = [sc[e] / denom * ROUTED_SCALE for e in range(NUM_EXPERTS)]
    scores_ref[...] = jnp.concatenate(w, axis=0)
    mask_ref[...] = jnp.concatenate(
        [sel[e].astype(jnp.int32) for e in range(NUM_EXPERTS)], axis=0
    )


def _routing(x, gate_weight, gate_bias):
    nblk = TOKENS // TBLK
    return pl.pallas_call(
        _routing_body,
        grid=(nblk,),
        in_specs=[
            pl.BlockSpec((TBLK, HIDDEN), lambda t: (t, 0)),
            pl.BlockSpec((NUM_EXPERTS, HIDDEN), lambda t: (0, 0)),
            pl.BlockSpec((NUM_EXPERTS, 1), lambda t: (0, 0)),
        ],
        out_specs=[
            pl.BlockSpec((NUM_EXPERTS, TBLK), lambda t: (0, t)),
            pl.BlockSpec((NUM_EXPERTS, TBLK), lambda t: (0, t)),
        ],
        out_shape=[
            jax.ShapeDtypeStruct((NUM_EXPERTS, TOKENS), jnp.float32),
            jax.ShapeDtypeStruct((NUM_EXPERTS, TOKENS), jnp.int32),
        ],
        interpret=_INTERP,
    )(x, gate_weight, gate_bias.reshape(NUM_EXPERTS, 1))


def _shared_body(x_ref, wg_ref, wu_ref, wd_ref, out_ref):
    x = x_ref[...]
    hg = jnp.dot(x, wg_ref[...], preferred_element_type=jnp.float32)
    hu = jnp.dot(x, wu_ref[...], preferred_element_type=jnp.float32)
    h = jax.nn.silu(hg) * hu
    out_ref[...] = jnp.dot(h, wd_ref[...], preferred_element_type=jnp.float32)


def _shared(x, wg, wu, wd):
    nblk = TOKENS // TBLK
    return pl.pallas_call(
        _shared_body,
        grid=(nblk,),
        in_specs=[
            pl.BlockSpec((TBLK, HIDDEN), lambda t: (t, 0)),
            pl.BlockSpec((HIDDEN, INTER), lambda t: (0, 0)),
            pl.BlockSpec((HIDDEN, INTER), lambda t: (0, 0)),
            pl.BlockSpec((INTER, HIDDEN), lambda t: (0, 0)),
        ],
        out_specs=pl.BlockSpec((TBLK, HIDDEN), lambda t: (t, 0)),
        out_shape=jax.ShapeDtypeStruct((TOKENS, HIDDEN), jnp.float32),
        interpret=_INTERP,
    )(x, wg, wu, wd)


def _iota16():
    return lax.iota(jnp.int32, 16)


def _splat_i(s):
    return jnp.full((16,), s, jnp.int32)


def _scan_gather_body(x_hbm, mask_hbm, scores_hbm,
                      xs_hbm, wsort_hbm, pos0_hbm, pos1_hbm, te_hbm,
                      mvm, svm, cntv, partials_v, partials_sh,
                      p0loc, p1loc, w0loc, w1loc,
                      p0sh, p1sh, w0sh, w1sh,
                      p0all, p1all, w0all, w1all,
                      stloc, wsloc, teloc, rowbuf, sem, wsem):
    cid = lax.axis_index("c")
    sid = lax.axis_index("s")
    wid = cid * NS + sid
    t0 = sid * TPS

    scope = jax.named_scope
    stages = []
    for e in range(NUM_EXPERTS):
        stages.append(pltpu.async_copy(mask_hbm.at[e, pl.ds(t0, TPS)],
                                       mvm.at[pl.ds(e * TPS, TPS)], sem.at[0]))
        stages.append(pltpu.async_copy(scores_hbm.at[e, pl.ds(t0, TPS)],
                                       svm.at[pl.ds(e * TPS, TPS)], sem.at[1]))
    for cp in stages:
        cp.wait()

    sA = scope("scanA"); sA.__enter__()
    it16 = _iota16()
    cnt_vec = jnp.zeros((16,), jnp.int32)
    for e in range(NUM_EXPERTS):
        acc = jnp.zeros((16,), jnp.int32)
        for ch in range(TPS // 16):
            acc = acc + mvm[pl.ds(e * TPS + ch * 16, 16)]
        cnt_e = jnp.sum(acc)
        cnt_vec = cnt_vec + jnp.where(it16 == e, cnt_e, 0)
    cntv[...] = cnt_vec
    pltpu.sync_copy(cntv, partials_sh.at[sid])
    plsc.subcore_barrier()

    pltpu.sync_copy(partials_sh, partials_v)
    base_vec = jnp.zeros((16,), jnp.int32)
    totc_vec = jnp.zeros((16,), jnp.int32)
    for s2 in range(NS):
        row = partials_v[s2]
        pred = jnp.full((16,), s2 < sid)
        base_vec = base_vec + jnp.where(pred, row, 0)
        totc_vec = totc_vec + row
    tiles_vec = (totc_vec + (B - 1)) // B
    cumt_vec = jnp.cumsum(tiles_vec)
    seg_vec = (cumt_vec - tiles_vec) * B

    sA.__exit__(None, None, None)
    sB = scope("scanB"); sB.__enter__()
    carry = [seg_vec[e] + base_vec[e] for e in range(NUM_EXPERTS)]
    for ch in range(TPS // 16):
        tokidx = it16 + (ch * 16)
        acc_k = jnp.zeros((16,), jnp.int32)
        p0v = jnp.zeros((16,), jnp.int32)
        p1v = jnp.zeros((16,), jnp.int32)
        w0v = jnp.zeros((16,), jnp.float32)
        w1v = jnp.zeros((16,), jnp.float32)
        for e in range(NUM_EXPERTS):
            mv = mvm[pl.ds(e * TPS + ch * 16, 16)]
            sv = svm[pl.ds(e * TPS + ch * 16, 16)]
            excl = jnp.cumsum(mv) - mv
            posv = excl + carry[e]
            selb = mv > 0
            first = selb & (acc_k == 0)
            second = selb & (acc_k == 1)
            p0v = jnp.where(first, posv, p0v)
            w0v = jnp.where(first, sv, w0v)
            p1v = jnp.where(second, posv, p1v)
            w1v = jnp.where(second, sv, w1v)
            acc_k = acc_k + mv
            carry[e] = carry[e] + jnp.sum(mv)
        p0loc[pl.ds(ch * 16, 16)] = p0v
        p1loc[pl.ds(ch * 16, 16)] = p1v
        w0loc[pl.ds(ch * 16, 16)] = w0v
        w1loc[pl.ds(ch * 16, 16)] = w1v

    pltpu.sync_copy(p0loc, p0sh.at[pl.ds(t0, TPS)])
    pltpu.sync_copy(p1loc, p1sh.at[pl.ds(t0, TPS)])
    pltpu.sync_copy(w0loc, w0sh.at[pl.ds(t0, TPS)])
    pltpu.sync_copy(w1loc, w1sh.at[pl.ds(t0, TPS)])

    @pl.when(cid == 0)
    def _():
        pltpu.sync_copy(p0loc, pos0_hbm.at[pl.ds(t0, TPS)])

    @pl.when(cid == 1)
    def _():
        pltpu.sync_copy(p1loc, pos1_hbm.at[pl.ds(t0, TPS)])

    @pl.when(wid == 0)
    def _():
        for ch in range(NTP // 16):
            jv = it16 + (ch * 16)
            te = jnp.zeros((16,), jnp.int32)
            for e in range(NUM_EXPERTS):
                te = te + (jv >= cumt_vec[e]).astype(jnp.int32)
            teloc[pl.ds(ch * 16, 16)] = jnp.minimum(te, NUM_EXPERTS - 1)
        pltpu.sync_copy(teloc, te_hbm)

    plsc.subcore_barrier()

    sB.__exit__(None, None, None)
    sC = scope("scanC"); sC.__enter__()
    pltpu.sync_copy(p0sh, p0all)
    pltpu.sync_copy(p1sh, p1all)
    pltpu.sync_copy(w0sh, w0all)
    pltpu.sync_copy(w1sh, w1all)

    r0 = wid * RPW
    zf = jnp.zeros((16,), jnp.float32)
    for ch in range(RPW // 16):
        stloc[pl.ds(ch * 16, 16)] = (it16 + (r0 + ch * 16)) & (TOKENS - 1)
        wsloc[pl.ds(ch * 16, 16)] = zf
    for ch in range(TOKENS // 16):
        sl = pl.ds(ch * 16, 16)
        tokv = it16 + (ch * 16)
        for pall, wall in ((p0all, w0all), (p1all, w1all)):
            pv = pall[sl] - r0
            wv = wall[sl]
            inb = (pv >= 0) & (pv < RPW)
            plsc.store_scatter(stloc, [pv], tokv, mask=inb)
            plsc.store_scatter(wsloc, [pv], wv, mask=inb)

    pltpu.sync_copy(wsloc, wsort_hbm.at[pl.ds(r0, RPW)])

    sC.__exit__(None, None, None)
    sD = scope("scanD"); sD.__enter__()
    nch = RPW // GCH
    gats = {}
    wbs = {}

    def _start(ch):
        buf_i = ch % NBUF
        idx = stloc.at[pl.ds(ch * GCH, GCH)]
        gats[buf_i] = pltpu.async_copy(x_hbm.at[idx], rowbuf.at[buf_i],
                                       sem.at[buf_i])

    for ch in range(min(NBUF - 1, nch)):
        _start(ch)
    for ch in range(nch):
        b = ch % NBUF
        if ch + NBUF - 1 < nch:
            nb = (ch + NBUF - 1) % NBUF
            if nb in wbs:
                wbs[nb].wait()
            _start(ch + NBUF - 1)
        gats[b].wait()
        _UNUSED = pltpu.async_copy(rowbuf.at[b],
                                  xs_hbm.at[pl.ds(r0 + ch * GCH, GCH)],
                                  wsem.at[b])
        wbs[b] = _UNUSED
    for b in wbs:
        wbs[b].wait()
    sD.__exit__(None, None, None)


def _scan_gather(x, mask2, scores2):
    mesh = plsc.VectorSubcoreMesh(core_axis_name="c", subcore_axis_name="s")
    kern = functools.partial(
        pl.kernel,
        mesh=mesh,
        compiler_params=pltpu.CompilerParams(needs_layout_passes=False),
        out_type=[
            jax.ShapeDtypeStruct((NR, HIDDEN), jnp.float32),
            jax.ShapeDtypeStruct((NR,), jnp.float32),
            jax.ShapeDtypeStruct((TOKENS,), jnp.int32),
            jax.ShapeDtypeStruct((TOKENS,), jnp.int32),
            jax.ShapeDtypeStruct((NTP,), jnp.int32),
        ],
        scratch_types=[
            pltpu.VMEM((TPS * NUM_EXPERTS,), jnp.int32),
            pltpu.VMEM((TPS * NUM_EXPERTS,), jnp.float32),
            pltpu.VMEM((16,), jnp.int32),
            pltpu.VMEM((NS, 16), jnp.int32),
            pltpu.VMEM_SHARED((NS, 16), jnp.int32),
            pltpu.VMEM((TPS,), jnp.int32),
            pltpu.VMEM((TPS,), jnp.int32),
            pltpu.VMEM((TPS,), jnp.float32),
            pltpu.VMEM((TPS,), jnp.float32),
            pltpu.VMEM_SHARED((TOKENS,), jnp.int32),
            pltpu.VMEM_SHARED((TOKENS,), jnp.int32),
            pltpu.VMEM_SHARED((TOKENS,), jnp.float32),
            pltpu.VMEM_SHARED((TOKENS,), jnp.float32),
            pltpu.VMEM((TOKENS,), jnp.int32),
            pltpu.VMEM((TOKENS,), jnp.int32),
            pltpu.VMEM((TOKENS,), jnp.float32),
            pltpu.VMEM((TOKENS,), jnp.float32),
            pltpu.VMEM((RPW,), jnp.int32),
            pltpu.VMEM((RPW,), jnp.float32),
            pltpu.VMEM((NTP,), jnp.int32),
            pltpu.VMEM((NBUF, GCH, HIDDEN), jnp.float32),
            pltpu.SemaphoreType.DMA((NBUF,)),
            pltpu.SemaphoreType.DMA((NBUF,)),
        ],
    )(_scan_gather_body)
    return kern(x, mask2, scores2)


def _grouped_body(te_ref, xs_ref, wg_ref, wu_ref, wd_ref, ws_ref, out_ref):
    x = xs_ref[...]
    hg = jnp.dot(x, wg_ref[0], preferred_element_type=jnp.float32)
    hu = jnp.dot(x, wu_ref[0], preferred_element_type=jnp.float32)
    h = jax.nn.silu(hg) * hu
    o = jnp.dot(h, wd_ref[0], preferred_element_type=jnp.float32)
    out_ref[...] = o * ws_ref[...]


def _grouped(xs, ewg, ewu, ewd, wsort2d, te):
    grid_spec = pltpu.PrefetchScalarGridSpec(
        num_scalar_prefetch=1,
        grid=(NT,),
        in_specs=[
            pl.BlockSpec((B, HIDDEN), lambda i, te: (i, 0)),
            pl.BlockSpec((1, HIDDEN, INTER), lambda i, te: (te[i], 0, 0)),
            pl.BlockSpec((1, HIDDEN, INTER), lambda i, te: (te[i], 0, 0)),
            pl.BlockSpec((1, INTER, HIDDEN), lambda i, te: (te[i], 0, 0)),
            pl.BlockSpec((B, 1), lambda i, te: (i, 0)),
        ],
        out_specs=pl.BlockSpec((B, HIDDEN), lambda i, te: (i, 0)),
    )
    return pl.pallas_call(
        _grouped_body,
        grid_spec=grid_spec,
        out_shape=jax.ShapeDtypeStruct((NR, HIDDEN), jnp.float32),
        interpret=_INTERP,
    )(te, xs, ewg, ewu, ewd, wsort2d)


def _combine_body(outs_hbm, pos0_hbm, pos1_hbm, shared_hbm, out_hbm,
                  p0v, p1v, rows0, rows1, shbuf, obuf, gsem, wsem):
    cid = lax.axis_index("c")
    sid = lax.axis_index("s")
    wid = cid * NS + sid
    t0 = wid * TPW
    nch = TPW // CT
    nsl = HIDDEN // 16

    pltpu.sync_copy(pos0_hbm.at[pl.ds(t0, TPW)], p0v)
    pltpu.sync_copy(pos1_hbm.at[pl.ds(t0, TPW)], p1v)

    gats = {}
    wbs = {}

    def _start(ch, b):
        idx0 = p0v.at[pl.ds(ch * CT, CT)]
        idx1 = p1v.at[pl.ds(ch * CT, CT)]
        gats[b] = (
            pltpu.async_copy(outs_hbm.at[idx0], rows0.at[b], gsem.at[b]),
            pltpu.async_copy(outs_hbm.at[idx1], rows1.at[b], gsem.at[b]),
            pltpu.async_copy(shared_hbm.at[pl.ds(t0 + ch * CT, CT)],
                             shbuf.at[b], gsem.at[b]),
        )

    _start(0, 0)
    for ch in range(nch):
        b = ch % 2
        nb = 1 - b
        if ch + 1 < nch:
            if nb in wbs:
                wbs[nb].wait()
            _start(ch + 1, nb)
        for cp in gats[b]:
            cp.wait()
        ob = obuf.at[b]
        r0b = rows0.at[b]
        r1b = rows1.at[b]
        shb = shbuf.at[b]

        @plsc.parallel_loop(0, CT * nsl, 1, unroll=8)
        def _(j):
            i = j // nsl
            sl = pl.ds((j % nsl) * 16, 16)
            ob[i, sl] = shb[i, sl] + r0b[i, sl] + r1b[i, sl]

        wbs[b] = pltpu.async_copy(obuf.at[b],
                                  out_hbm.at[pl.ds(t0 + ch * CT, CT)],
                                  wsem.at[b])
    for b in wbs:
        wbs[b].wait()


def _combine(outs, pos0, pos1, shared_out):
    mesh = plsc.VectorSubcoreMesh(core_axis_name="c", subcore_axis_name="s")
    kern = functools.partial(
        pl.kernel,
        mesh=mesh,
        compiler_params=pltpu.CompilerParams(needs_layout_passes=False),
        out_type=jax.ShapeDtypeStruct((TOKENS, HIDDEN), jnp.float32),
        scratch_types=[
            pltpu.VMEM((TPW,), jnp.int32),
            pltpu.VMEM((TPW,), jnp.int32),
            pltpu.VMEM((2, CT, HIDDEN), jnp.float32),
            pltpu.VMEM((2, CT, HIDDEN), jnp.float32),
            pltpu.VMEM((2, CT, HIDDEN), jnp.float32),
            pltpu.VMEM((2, CT, HIDDEN), jnp.float32),
            pltpu.SemaphoreType.DMA((2,)),
            pltpu.SemaphoreType.DMA((2,)),
        ],
    )(_combine_body)
    return kern(outs, pos0, pos1, shared_out)


def kernel(hidden_states, gate_weight, gate_bias, expert_w_gate, expert_w_up,
           expert_w_down, shared_w_gate, shared_w_up, shared_w_down):
    x = hidden_states.astype(jnp.float32)
    scores, mask = _routing(x, gate_weight, gate_bias)
    xs, wsort, pos0, pos1, te = _scan_gather(x, mask, scores)
    outs = _grouped(xs, expert_w_gate, expert_w_up, expert_w_down,
                    wsort.reshape(NR, 1), te)
    shared_out = _shared(x, shared_w_gate, shared_w_up, shared_w_down)
    return _combine(outs, pos0, pos1, shared_out)


# speedup vs baseline: 1.7874x; 1.0096x over previous
import functools

import jax
import jax.numpy as jnp
from jax import lax
from jax.experimental import pallas as pl
from jax.experimental.pallas import tpu as pltpu
from jax.experimental.pallas import tpu_sc as plsc

NUM_EXPERTS = 8
TOP_K = 2
HIDDEN = 1024
INTER = 512
N_GROUP = 4
GSZ = NUM_EXPERTS // N_GROUP
TOPK_GROUP = 2
ROUTED_SCALE = 2.5
TOKENS = 2048

TBLK = 256
B = 128
NT = 40
NTP = 48
NR = NT * B

NC = 2
NS = 16
NW = NC * NS
TPW = TOKENS // NW
TPS = TOKENS // NS
RPW = NR // NW
GCH = 16
NBUF = 4
CT = 8
PKW = HIDDEN // 2


def _pack_bf16_pairs(o):
    lo = jax.lax.bitcast_convert_type(o[:, :PKW].astype(jnp.bfloat16), jnp.uint16)
    hi = jax.lax.bitcast_convert_type(o[:, PKW:].astype(jnp.bfloat16), jnp.uint16)
    w = lo.astype(jnp.uint32) | (hi.astype(jnp.uint32) << 16)
    return jax.lax.bitcast_convert_type(w, jnp.int32)

_INTERP = False


def _routing_body(x_ref, gw_ref, gb_ref, scores_ref, mask_ref):
    logits = lax.dot_general(gw_ref[...], x_ref[...],
                             dimension_numbers=(((1,), (1,)), ((), ())),
                             preferred_element_type=jnp.float32)
    s = jax.nn.sigmoid(logits)
    swb = s + gb_ref[...]

    row = lambda a, i: a[i : i + 1, :]
    g = [sum(row(swb, gi * GSZ + j) for j in range(GSZ)) for gi in range(N_GROUP)]
    gsel = []
    for gi in range(N_GROUP):
        rank = jnp.zeros_like(g[gi])
        for gj in range(N_GROUP):
            if gj == gi:
                continue
            beats = g[gj] > g[gi]
            if gj < gi:
                beats = beats | (g[gj] == g[gi])
            rank = rank + beats.astype(jnp.float32)
        gsel.append(rank < TOPK_GROUP)
    swbm = [jnp.where(gsel[e // GSZ], row(swb, e), 0.0) for e in range(NUM_EXPERTS)]
    sel = []
    for e in range(NUM_EXPERTS):
        rank = jnp.zeros_like(swbm[e])
        for e2 in range(NUM_EXPERTS):
            if e2 == e:
                continue
            beats = swbm[e2] > swbm[e]
            if e2 < e:
                beats = beats | (swbm[e2] == swbm[e])
            rank = rank + beats.astype(jnp.float32)
        sel.append(rank < TOP_K)
    sc = [jnp.where(sel[e], row(s, e), 0.0) for e in range(NUM_EXPERTS)]
    denom = sum(sc) + 1e-20
    w = [sc[e] / denom * ROUTED_SCALE for e in range(NUM_EXPERTS)]
    scores_ref[...] = jnp.concatenate(w, axis=0)
    mask_ref[...] = jnp.concatenate(
        [sel[e].astype(jnp.int32) for e in range(NUM_EXPERTS)], axis=0
    )


def _routing(x, gate_weight, gate_bias):
    nblk = TOKENS // TBLK
    return pl.pallas_call(
        _routing_body,
        grid=(nblk,),
        in_specs=[
            pl.BlockSpec((TBLK, HIDDEN), lambda t: (t, 0)),
            pl.BlockSpec((NUM_EXPERTS, HIDDEN), lambda t: (0, 0)),
            pl.BlockSpec((NUM_EXPERTS, 1), lambda t: (0, 0)),
        ],
        out_specs=[
            pl.BlockSpec((NUM_EXPERTS, TBLK), lambda t: (0, t)),
            pl.BlockSpec((NUM_EXPERTS, TBLK), lambda t: (0, t)),
        ],
        out_shape=[
            jax.ShapeDtypeStruct((NUM_EXPERTS, TOKENS), jnp.float32),
            jax.ShapeDtypeStruct((NUM_EXPERTS, TOKENS), jnp.int32),
        ],
        interpret=_INTERP,
    )(x, gate_weight, gate_bias.reshape(NUM_EXPERTS, 1))


def _shared_body(x_ref, wg_ref, wu_ref, wd_ref, out_ref):
    x = x_ref[...]
    hg = jnp.dot(x, wg_ref[...], preferred_element_type=jnp.float32)
    hu = jnp.dot(x, wu_ref[...], preferred_element_type=jnp.float32)
    h = jax.nn.silu(hg) * hu
    o = jnp.dot(h, wd_ref[...], preferred_element_type=jnp.float32)
    out_ref[...] = _pack_bf16_pairs(o)


def _shared(x, wg, wu, wd):
    nblk = TOKENS // TBLK
    return pl.pallas_call(
        _shared_body,
        grid=(nblk,),
        in_specs=[
            pl.BlockSpec((TBLK, HIDDEN), lambda t: (t, 0)),
            pl.BlockSpec((HIDDEN, INTER), lambda t: (0, 0)),
            pl.BlockSpec((HIDDEN, INTER), lambda t: (0, 0)),
            pl.BlockSpec((INTER, HIDDEN), lambda t: (0, 0)),
        ],
        out_specs=pl.BlockSpec((TBLK, PKW), lambda t: (t, 0)),
        out_shape=jax.ShapeDtypeStruct((TOKENS, PKW), jnp.int32),
        interpret=_INTERP,
    )(x, wg, wu, wd)


def _iota16():
    return lax.iota(jnp.int32, 16)


def _splat_i(s):
    return jnp.full((16,), s, jnp.int32)


def _scan_gather_body(x_hbm, mask_hbm, scores_hbm,
                      xs_hbm, wsort_hbm, pos0_hbm, pos1_hbm, te_hbm,
                      mvm, svm, cntv, partials_v, partials_sh,
                      p0loc, p1loc, w0loc, w1loc,
                      p0sh, p1sh, w0sh, w1sh,
                      p0all, p1all, w0all, w1all,
                      stloc, wsloc, teloc, rowbuf, sem, wsem):
    cid = lax.axis_index("c")
    sid = lax.axis_index("s")
    wid = cid * NS + sid
    t0 = sid * TPS

    scope = jax.named_scope
    stages = []
    for e in range(NUM_EXPERTS):
        stages.append(pltpu.async_copy(mask_hbm.at[e, pl.ds(t0, TPS)],
                                       mvm.at[pl.ds(e * TPS, TPS)], sem.at[0]))
        stages.append(pltpu.async_copy(scores_hbm.at[e, pl.ds(t0, TPS)],
                                       svm.at[pl.ds(e * TPS, TPS)], sem.at[1]))
    for cp in stages:
        cp.wait()

    sA = scope("scanA"); sA.__enter__()
    it16 = _iota16()
    cnt_vec = jnp.zeros((16,), jnp.int32)
    for e in range(NUM_EXPERTS):
        acc = jnp.zeros((16,), jnp.int32)
        for ch in range(TPS // 16):
            acc = acc + mvm[pl.ds(e * TPS + ch * 16, 16)]
        cnt_e = jnp.sum(acc)
        cnt_vec = cnt_vec + jnp.where(it16 == e, cnt_e, 0)
    cntv[...] = cnt_vec
    pltpu.sync_copy(cntv, partials_sh.at[sid])
    plsc.subcore_barrier()

    pltpu.sync_copy(partials_sh, partials_v)
    base_vec = jnp.zeros((16,), jnp.int32)
    totc_vec = jnp.zeros((16,), jnp.int32)
    for s2 in range(NS):
        row = partials_v[s2]
        pred = jnp.full((16,), s2 < sid)
        base_vec = base_vec + jnp.where(pred, row, 0)
        totc_vec = totc_vec + row
    tiles_vec = (totc_vec + (B - 1)) // B
    cumt_vec = jnp.cumsum(tiles_vec)
    seg_vec = (cumt_vec - tiles_vec) * B

    sA.__exit__(None, None, None)
    sB = scope("scanB"); sB.__enter__()
    carry = [seg_vec[e] + base_vec[e] for e in range(NUM_EXPERTS)]
    for ch in range(TPS // 16):
        tokidx = it16 + (ch * 16)
        acc_k = jnp.zeros((16,), jnp.int32)
        p0v = jnp.zeros((16,), jnp.int32)
        p1v = jnp.zeros((16,), jnp.int32)
        w0v = jnp.zeros((16,), jnp.float32)
        w1v = jnp.zeros((16,), jnp.float32)
        for e in range(NUM_EXPERTS):
            mv = mvm[pl.ds(e * TPS + ch * 16, 16)]
            sv = svm[pl.ds(e * TPS + ch * 16, 16)]
            excl = jnp.cumsum(mv) - mv
            posv = excl + carry[e]
            selb = mv > 0
            first = selb & (acc_k == 0)
            second = selb & (acc_k == 1)
            p0v = jnp.where(first, posv, p0v)
            w0v = jnp.where(first, sv, w0v)
            p1v = jnp.where(second, posv, p1v)
            w1v = jnp.where(second, sv, w1v)
            acc_k = acc_k + mv
            carry[e] = carry[e] + jnp.sum(mv)
        p0loc[pl.ds(ch * 16, 16)] = p0v
        p1loc[pl.ds(ch * 16, 16)] = p1v
        w0loc[pl.ds(ch * 16, 16)] = w0v
        w1loc[pl.ds(ch * 16, 16)] = w1v

    pltpu.sync_copy(p0loc, p0sh.at[pl.ds(t0, TPS)])
    pltpu.sync_copy(p1loc, p1sh.at[pl.ds(t0, TPS)])
    pltpu.sync_copy(w0loc, w0sh.at[pl.ds(t0, TPS)])
    pltpu.sync_copy(w1loc, w1sh.at[pl.ds(t0, TPS)])

    @pl.when(cid == 0)
    def _():
        pltpu.sync_copy(p0loc, pos0_hbm.at[pl.ds(t0, TPS)])

    @pl.when(cid == 1)
    def _():
        pltpu.sync_copy(p1loc, pos1_hbm.at[pl.ds(t0, TPS)])

    @pl.when(wid == 0)
    def _():
        for ch in range(NTP // 16):
            jv = it16 + (ch * 16)
            te = jnp.zeros((16,), jnp.int32)
            for e in range(NUM_EXPERTS):
                te = te + (jv >= cumt_vec[e]).astype(jnp.int32)
            teloc[pl.ds(ch * 16, 16)] = jnp.minimum(te, NUM_EXPERTS - 1)
        pltpu.sync_copy(teloc, te_hbm)

    plsc.subcore_barrier()

    sB.__exit__(None, None, None)
    sC = scope("scanC"); sC.__enter__()
    pltpu.sync_copy(p0sh, p0all)
    pltpu.sync_copy(p1sh, p1all)
    pltpu.sync_copy(w0sh, w0all)
    pltpu.sync_copy(w1sh, w1all)

    r0 = wid * RPW
    zf = jnp.zeros((16,), jnp.float32)
    for ch in range(RPW // 16):
        stloc[pl.ds(ch * 16, 16)] = (it16 + (r0 + ch * 16)) & (TOKENS - 1)
        wsloc[pl.ds(ch * 16, 16)] = zf
    for ch in range(TOKENS // 16):
        sl = pl.ds(ch * 16, 16)
        tokv = it16 + (ch * 16)
        for pall, wall in ((p0all, w0all), (p1all, w1all)):
            pv = pall[sl] - r0
            wv = wall[sl]
            inb = (pv >= 0) & (pv < RPW)
            plsc.store_scatter(stloc, [pv], tokv, mask=inb)
            plsc.store_scatter(wsloc, [pv], wv, mask=inb)

    pltpu.sync_copy(wsloc, wsort_hbm.at[pl.ds(r0, RPW)])

    sC.__exit__(None, None, None)
    sD = scope("scanD"); sD.__enter__()
    nch = RPW // GCH
    gats = {}
    wbs = {}

    def _start(ch):
        buf_i = ch % NBUF
        idx = stloc.at[pl.ds(ch * GCH, GCH)]
        gats[buf_i] = pltpu.async_copy(x_hbm.at[idx], rowbuf.at[buf_i],
                                       sem.at[buf_i])

    for ch in range(min(NBUF - 1, nch)):
        _start(ch)
    for ch in range(nch):
        b = ch % NBUF
        if ch + NBUF - 1 < nch:
            nb = (ch + NBUF - 1) % NBUF
            if nb in wbs:
                wbs[nb].wait()
            _start(ch + NBUF - 1)
        gats[b].wait()
        _UNUSED = pltpu.async_copy(rowbuf.at[b],
                                  xs_hbm.at[pl.ds(r0 + ch * GCH, GCH)],
                                  wsem.at[b])
        wbs[b] = _UNUSED
    for b in wbs:
        wbs[b].wait()
    sD.__exit__(None, None, None)


def _scan_gather(x, mask2, scores2):
    mesh = plsc.VectorSubcoreMesh(core_axis_name="c", subcore_axis_name="s")
    kern = functools.partial(
        pl.kernel,
        mesh=mesh,
        compiler_params=pltpu.CompilerParams(needs_layout_passes=False),
        out_type=[
            jax.ShapeDtypeStruct((NR, HIDDEN), jnp.float32),
            jax.ShapeDtypeStruct((NR,), jnp.float32),
            jax.ShapeDtypeStruct((TOKENS,), jnp.int32),
            jax.ShapeDtypeStruct((TOKENS,), jnp.int32),
            jax.ShapeDtypeStruct((NTP,), jnp.int32),
        ],
        scratch_types=[
            pltpu.VMEM((TPS * NUM_EXPERTS,), jnp.int32),
            pltpu.VMEM((TPS * NUM_EXPERTS,), jnp.float32),
            pltpu.VMEM((16,), jnp.int32),
            pltpu.VMEM((NS, 16), jnp.int32),
            pltpu.VMEM_SHARED((NS, 16), jnp.int32),
            pltpu.VMEM((TPS,), jnp.int32),
            pltpu.VMEM((TPS,), jnp.int32),
            pltpu.VMEM((TPS,), jnp.float32),
            pltpu.VMEM((TPS,), jnp.float32),
            pltpu.VMEM_SHARED((TOKENS,), jnp.int32),
            pltpu.VMEM_SHARED((TOKENS,), jnp.int32),
            pltpu.VMEM_SHARED((TOKENS,), jnp.float32),
            pltpu.VMEM_SHARED((TOKENS,), jnp.float32),
            pltpu.VMEM((TOKENS,), jnp.int32),
            pltpu.VMEM((TOKENS,), jnp.int32),
            pltpu.VMEM((TOKENS,), jnp.float32),
            pltpu.VMEM((TOKENS,), jnp.float32),
            pltpu.VMEM((RPW,), jnp.int32),
            pltpu.VMEM((RPW,), jnp.float32),
            pltpu.VMEM((NTP,), jnp.int32),
            pltpu.VMEM((NBUF, GCH, HIDDEN), jnp.float32),
            pltpu.SemaphoreType.DMA((NBUF,)),
            pltpu.SemaphoreType.DMA((NBUF,)),
        ],
    )(_scan_gather_body)
    return kern(x, mask2, scores2)


def _grouped_body(te_ref, xs_ref, wg_ref, wu_ref, wd_ref, ws_ref, out_ref):
    x = xs_ref[...]
    hg = jnp.dot(x, wg_ref[0], preferred_element_type=jnp.float32)
    hu = jnp.dot(x, wu_ref[0], preferred_element_type=jnp.float32)
    h = jax.nn.silu(hg) * hu
    o = jnp.dot(h, wd_ref[0], preferred_element_type=jnp.float32)
    out_ref[...] = _pack_bf16_pairs(o * ws_ref[...])


def _grouped(xs, ewg, ewu, ewd, wsort2d, te):
    grid_spec = pltpu.PrefetchScalarGridSpec(
        num_scalar_prefetch=1,
        grid=(NT,),
        in_specs=[
            pl.BlockSpec((B, HIDDEN), lambda i, te: (i, 0)),
            pl.BlockSpec((1, HIDDEN, INTER), lambda i, te: (te[i], 0, 0)),
            pl.BlockSpec((1, HIDDEN, INTER), lambda i, te: (te[i], 0, 0)),
            pl.BlockSpec((1, INTER, HIDDEN), lambda i, te: (te[i], 0, 0)),
            pl.BlockSpec((B, 1), lambda i, te: (i, 0)),
        ],
        out_specs=pl.BlockSpec((B, PKW), lambda i, te: (i, 0)),
    )
    return pl.pallas_call(
        _grouped_body,
        grid_spec=grid_spec,
        out_shape=jax.ShapeDtypeStruct((NR, PKW), jnp.int32),
        interpret=_INTERP,
    )(te, xs, ewg, ewu, ewd, wsort2d)


def _combine_body(outs_hbm, pos0_hbm, pos1_hbm, shared_hbm, out_hbm,
                  p0v, p1v, rows0, rows1, shbuf, obuf, gsem, wsem):
    cid = lax.axis_index("c")
    sid = lax.axis_index("s")
    wid = cid * NS + sid
    t0 = wid * TPW
    nch = TPW // CT
    nsl = PKW // 16

    pltpu.sync_copy(pos0_hbm.at[pl.ds(t0, TPW)], p0v)
    pltpu.sync_copy(pos1_hbm.at[pl.ds(t0, TPW)], p1v)

    gats = {}
    wbs = {}

    def _start(ch, b):
        idx0 = p0v.at[pl.ds(ch * CT, CT)]
        idx1 = p1v.at[pl.ds(ch * CT, CT)]
        gats[b] = (
            pltpu.async_copy(outs_hbm.at[idx0], rows0.at[b], gsem.at[b]),
            pltpu.async_copy(outs_hbm.at[idx1], rows1.at[b], gsem.at[b]),
            pltpu.async_copy(shared_hbm.at[pl.ds(t0 + ch * CT, CT)],
                             shbuf.at[b], gsem.at[b]),
        )

    _start(0, 0)
    for ch in range(nch):
        b = ch % 2
        nb = 1 - b
        if ch + 1 < nch:
            if nb in wbs:
                wbs[nb].wait()
            _start(ch + 1, nb)
        for cp in gats[b]:
            cp.wait()
        ob = obuf.at[b]
        r0b = rows0.at[b]
        r1b = rows1.at[b]
        shb = shbuf.at[b]

        @plsc.parallel_loop(0, CT * nsl, 1, unroll=8)
        def _(j):
            i = j // nsl
            sl = pl.ds((j % nsl) * 16, 16)
            a = plsc.bitcast(shb[i, sl], jnp.bfloat16)
            b2 = plsc.bitcast(r0b[i, sl], jnp.bfloat16)
            c2 = plsc.bitcast(r1b[i, sl], jnp.bfloat16)
            ob[i, sl] = plsc.bitcast(a + b2 + c2, jnp.int32)

        wbs[b] = pltpu.async_copy(obuf.at[b],
                                  out_hbm.at[pl.ds(t0 + ch * CT, CT)],
                                  wsem.at[b])
    for b in wbs:
        wbs[b].wait()


def _combine(outs, pos0, pos1, shared_out):
    mesh = plsc.VectorSubcoreMesh(core_axis_name="c", subcore_axis_name="s")
    kern = functools.partial(
        pl.kernel,
        mesh=mesh,
        compiler_params=pltpu.CompilerParams(needs_layout_passes=False),
        out_type=jax.ShapeDtypeStruct((TOKENS, PKW), jnp.int32),
        scratch_types=[
            pltpu.VMEM((TPW,), jnp.int32),
            pltpu.VMEM((TPW,), jnp.int32),
            pltpu.VMEM((2, CT, PKW), jnp.int32),
            pltpu.VMEM((2, CT, PKW), jnp.int32),
            pltpu.VMEM((2, CT, PKW), jnp.int32),
            pltpu.VMEM((2, CT, PKW), jnp.int32),
            pltpu.SemaphoreType.DMA((2,)),
            pltpu.SemaphoreType.DMA((2,)),
        ],
    )(_combine_body)
    return kern(outs, pos0, pos1, shared_out)


def kernel(hidden_states, gate_weight, gate_bias, expert_w_gate, expert_w_up,
           expert_w_down, shared_w_gate, shared_w_up, shared_w_down):
    x = hidden_states.astype(jnp.float32)
    scores, mask = _routing(x, gate_weight, gate_bias)
    xs, wsort, pos0, pos1, te = _scan_gather(x, mask, scores)
    outs = _grouped(xs, expert_w_gate, expert_w_up, expert_w_down,
                    wsort.reshape(NR, 1), te)
    shared_out = _shared(x, shared_w_gate, shared_w_up, shared_w_down)
    packed = _combine(outs, pos0, pos1, shared_out)
    u = jax.lax.bitcast_convert_type(packed, jnp.uint32)
    lo = jax.lax.bitcast_convert_type((u & 0xFFFF).astype(jnp.uint16),
                                      jnp.bfloat16).astype(jnp.float32)
    hi = jax.lax.bitcast_convert_type((u >> 16).astype(jnp.uint16),
                                      jnp.bfloat16).astype(jnp.float32)
    return jnp.concatenate([lo, hi], axis=1)
